# baseline (speedup 1.0000x reference)
import jax
import jax.numpy as jnp
from jax import lax
from jax.experimental import pallas as pl
from jax.experimental.pallas import tpu as pltpu
from jax.experimental.pallas import tpu_sc as plsc

N_NODES = 10000
D = 128
N_CLASSES = 40

NC = 2
NS = 16
CHUNK = 128
CPT = 80
GCH = 16
E_PAD = NC * NS * CPT * CHUNK
RPT = 632
N_PAD = NS * RPT
ROW_BLK = RPT
GRID = N_PAD // ROW_BLK

_MESH = plsc.VectorSubcoreMesh(core_axis_name="c", subcore_axis_name="s")


def _agg_body(feat, srci, dsti, zer, acc_out, src_v, dst_v, rows_v, acc_s, sem):
  c = lax.axis_index("c")
  s = lax.axis_index("s")
  w = c * NS + s
  r0 = s * RPT
  pltpu.sync_copy(zer, rows_v.at[0])
  for k in range(4):
    pltpu.sync_copy(rows_v.at[0], acc_s.at[pl.ds(r0 + k * CHUNK, CHUNK)])
  pltpu.sync_copy(rows_v.at[0, pl.ds(0, RPT - 4 * CHUNK)],
                  acc_s.at[pl.ds(r0 + 4 * CHUNK, RPT - 4 * CHUNK)])
  plsc.subcore_barrier()

  def group(g, carry):
    base = w * CPT + g * GCH
    pltpu.sync_copy(srci.at[pl.ds(base, GCH)], src_v)
    pltpu.sync_copy(dsti.at[pl.ds(base, GCH)], dst_v)
    pltpu.async_copy(feat.at[src_v.at[0]], rows_v.at[0], sem)

    def step(b, c2):
      p = lax.rem(b, 2)
      pltpu.make_async_copy(feat.at[src_v.at[b]], rows_v.at[p], sem).wait()

      @pl.when(b + 1 < GCH)
      def _():
        pltpu.async_copy(feat.at[src_v.at[b + 1]], rows_v.at[1 - p], sem)

      pltpu.sync_copy(rows_v.at[p], acc_s.at[dst_v.at[b]], add=True)
      return c2

    lax.fori_loop(0, GCH, step, 0)
    return carry

  lax.fori_loop(0, CPT // GCH, group, 0)

  plsc.subcore_barrier()
  ob = c * N_PAD + r0
  for k in range(4):
    pltpu.sync_copy(acc_s.at[pl.ds(r0 + k * CHUNK, CHUNK)], rows_v.at[0])
    pltpu.sync_copy(rows_v.at[0], acc_out.at[pl.ds(ob + k * CHUNK, CHUNK)])
  pltpu.sync_copy(acc_s.at[pl.ds(r0 + 4 * CHUNK, RPT - 4 * CHUNK)],
                  rows_v.at[0, pl.ds(0, RPT - 4 * CHUNK)])
  pltpu.sync_copy(rows_v.at[0, pl.ds(0, RPT - 4 * CHUNK)],
                  acc_out.at[pl.ds(ob + 4 * CHUNK, RPT - 4 * CHUNK)])


_agg = pl.kernel(
    _agg_body, mesh=_MESH,
    out_type=[jax.ShapeDtypeStruct((NC * N_PAD, D), jnp.float32)],
    scratch_types=[
        pltpu.VMEM((GCH, CHUNK), jnp.int32),
        pltpu.VMEM((GCH, CHUNK), jnp.int32),
        pltpu.VMEM((2, CHUNK, D), jnp.float32),
        pltpu.VMEM_SHARED((N_PAD, D), jnp.float32),
        pltpu.SemaphoreType.DMA,
    ])


def _count_body(dsti, zer, ones, cnt_out, dst_v, ones_v, cnt_s):
  c = lax.axis_index("c")
  s = lax.axis_index("s")
  w = c * NS + s
  r0 = s * RPT
  pltpu.sync_copy(zer, cnt_s.at[pl.ds(r0, RPT)])
  pltpu.sync_copy(ones, ones_v)
  plsc.subcore_barrier()

  def group(g, carry):
    base = w * CPT + g * GCH
    pltpu.sync_copy(dsti.at[pl.ds(base, GCH)], dst_v)

    def step(b, c2):
      pltpu.sync_copy(ones_v, cnt_s.at[dst_v.at[b]], add=True)
      return c2

    lax.fori_loop(0, GCH, step, 0)
    return carry

  lax.fori_loop(0, CPT // GCH, group, 0)

  plsc.subcore_barrier()
  ob = c * N_PAD + r0
  pltpu.sync_copy(cnt_s.at[pl.ds(r0, RPT)], cnt_out.at[pl.ds(ob, RPT)])


_count = pl.kernel(
    _count_body, mesh=_MESH,
    out_type=[jax.ShapeDtypeStruct((NC * N_PAD, D), jnp.float32)],
    scratch_types=[
        pltpu.VMEM((GCH, CHUNK), jnp.int32),
        pltpu.VMEM((CHUNK, D), jnp.float32),
        pltpu.VMEM_SHARED((N_PAD, D), jnp.float32),
    ])


def _dense1(acc, cnt, x, wl, b, wr):
  def body(a0, a1, c0, c1, x_ref, wl_ref, b_ref, wr_ref, h_ref):
    sums = a0[...] + a1[...]
    counts = jnp.maximum(c0[:, 0:1] + c1[:, 0:1], 1.0)
    agg = sums / counts
    h = (jnp.dot(agg, wl_ref[...], preferred_element_type=jnp.float32)
         + b_ref[...]
         + jnp.dot(x_ref[...], wr_ref[...], preferred_element_type=jnp.float32))
    h_ref[...] = jnp.maximum(h, 0.0)

  return pl.pallas_call(
      body,
      grid=(GRID,),
      in_specs=[
          pl.BlockSpec((ROW_BLK, D), lambda i: (i, 0)),
          pl.BlockSpec((ROW_BLK, D), lambda i: (i + GRID, 0)),
          pl.BlockSpec((ROW_BLK, D), lambda i: (i, 0)),
          pl.BlockSpec((ROW_BLK, D), lambda i: (i + GRID, 0)),
          pl.BlockSpec((ROW_BLK, D), lambda i: (i, 0)),
          pl.BlockSpec((D, D), lambda i: (0, 0)),
          pl.BlockSpec((1, D), lambda i: (0, 0)),
          pl.BlockSpec((D, D), lambda i: (0, 0)),
      ],
      out_specs=pl.BlockSpec((ROW_BLK, D), lambda i: (i, 0)),
      out_shape=jax.ShapeDtypeStruct((N_PAD, D), jnp.float32),
  )(acc, acc, cnt, cnt, x, wl, b, wr)


def _dense2(acc, cnt, h, wl, b, wr):
  def body(a0, a1, c0, c1, h_ref, wl_ref, b_ref, wr_ref, o_ref):
    sums = a0[...] + a1[...]
    counts = jnp.maximum(c0[:, 0:1] + c1[:, 0:1], 1.0)
    agg = sums / counts
    logits = (jnp.dot(agg, wl_ref[...], preferred_element_type=jnp.float32)
              + b_ref[...]
              + jnp.dot(h_ref[...], wr_ref[...], preferred_element_type=jnp.float32))
    m = jnp.max(logits, axis=1, keepdims=True)
    lse = jnp.log(jnp.sum(jnp.exp(logits - m), axis=1, keepdims=True))
    o_ref[...] = logits - m - lse

  return pl.pallas_call(
      body,
      grid=(GRID,),
      in_specs=[
          pl.BlockSpec((ROW_BLK, D), lambda i: (i, 0)),
          pl.BlockSpec((ROW_BLK, D), lambda i: (i + GRID, 0)),
          pl.BlockSpec((ROW_BLK, D), lambda i: (i, 0)),
          pl.BlockSpec((ROW_BLK, D), lambda i: (i + GRID, 0)),
          pl.BlockSpec((ROW_BLK, D), lambda i: (i, 0)),
          pl.BlockSpec((D, N_CLASSES), lambda i: (0, 0)),
          pl.BlockSpec((1, N_CLASSES), lambda i: (0, 0)),
          pl.BlockSpec((D, N_CLASSES), lambda i: (0, 0)),
      ],
      out_specs=pl.BlockSpec((ROW_BLK, N_CLASSES), lambda i: (i, 0)),
      out_shape=jax.ShapeDtypeStruct((N_PAD, N_CLASSES), jnp.float32),
  )(acc, acc, cnt, cnt, h, wl, b, wr)


def kernel(x, edge_index, W1l, b1l, W1r, W2l, b2l, W2r):
  src = edge_index[0].astype(jnp.int32)
  dst = edge_index[1].astype(jnp.int32)
  pad = E_PAD - src.shape[0]
  src2 = jnp.concatenate([src, jnp.zeros((pad,), jnp.int32)]).reshape(-1, CHUNK)
  dst2 = jnp.concatenate([dst, jnp.full((pad,), N_NODES, jnp.int32)]).reshape(-1, CHUNK)
  x_pad = jnp.pad(x, ((0, N_PAD - N_NODES), (0, 0)))
  zer = jnp.zeros((RPT, D), jnp.float32)
  zer128 = jnp.zeros((CHUNK, D), jnp.float32)
  ones = jnp.ones((CHUNK, D), jnp.float32)

  (cnt,) = _count(dst2, zer, ones)
  (acc1,) = _agg(x_pad, src2, dst2, zer128)
  h = _dense1(acc1, cnt, x_pad, W1l.T, b1l.reshape(1, D), W1r.T)
  (acc2,) = _agg(h, src2, dst2, zer128)
  out = _dense2(acc2, cnt, h, W2l.T, b2l.reshape(1, N_CLASSES), W2r.T)
  return out[:N_NODES]


# speedup vs baseline: 1.0461x; 1.0461x over previous
import jax
import jax.numpy as jnp
from jax import lax
from jax.experimental import pallas as pl
from jax.experimental.pallas import tpu as pltpu
from jax.experimental.pallas import tpu_sc as plsc

N_NODES = 10000
D = 128
N_CLASSES = 40

NC = 2
NS = 16
CHUNK = 128
CPT = 80
GCH = 16
E_PAD = NC * NS * CPT * CHUNK
RPT = 632
N_PAD = NS * RPT
ROW_BLK = RPT
GRID = N_PAD // ROW_BLK

_MESH = plsc.VectorSubcoreMesh(core_axis_name="c", subcore_axis_name="s")


def _agg_body(feat, srci, dsti, zer, acc_out, src_v, dst_v, rows_v, acc_s, sem):
  c = lax.axis_index("c")
  s = lax.axis_index("s")
  w = c * NS + s
  r0 = s * RPT
  pltpu.sync_copy(zer, acc_s.at[pl.ds(r0, RPT)])
  plsc.subcore_barrier()

  def group(g, carry):
    base = w * CPT + g * GCH
    pltpu.sync_copy(srci.at[pl.ds(base, GCH)], src_v)
    pltpu.sync_copy(dsti.at[pl.ds(base, GCH)], dst_v)
    pltpu.async_copy(feat.at[src_v.at[0]], rows_v.at[0], sem)

    def step(b, c2):
      p = lax.rem(b, 2)
      pltpu.make_async_copy(feat.at[src_v.at[b]], rows_v.at[p], sem).wait()

      @pl.when(b + 1 < GCH)
      def _():
        pltpu.async_copy(feat.at[src_v.at[b + 1]], rows_v.at[1 - p], sem)

      pltpu.sync_copy(rows_v.at[p], acc_s.at[dst_v.at[b]], add=True)
      return c2

    lax.fori_loop(0, GCH, step, 0)
    return carry

  lax.fori_loop(0, CPT // GCH, group, 0)

  plsc.subcore_barrier()
  ob = c * N_PAD + r0
  pltpu.sync_copy(acc_s.at[pl.ds(r0, RPT)], acc_out.at[pl.ds(ob, RPT)])


_agg = pl.kernel(
    _agg_body, mesh=_MESH,
    out_type=[jax.ShapeDtypeStruct((NC * N_PAD, D), jnp.float32)],
    scratch_types=[
        pltpu.VMEM((GCH, CHUNK), jnp.int32),
        pltpu.VMEM((GCH, CHUNK), jnp.int32),
        pltpu.VMEM((2, CHUNK, D), jnp.float32),
        pltpu.VMEM_SHARED((N_PAD, D), jnp.float32),
        pltpu.SemaphoreType.DMA,
    ])


def _count_body(dsti, zer, ones, cnt_out, dst_v, ones_v, cnt_s, sem):
  c = lax.axis_index("c")
  s = lax.axis_index("s")
  w = c * NS + s
  r0 = s * RPT
  pltpu.sync_copy(zer, cnt_s.at[pl.ds(r0, RPT)])
  pltpu.sync_copy(ones, ones_v)
  plsc.subcore_barrier()
  ngrp = CPT // GCH
  pltpu.sync_copy(dsti.at[pl.ds(w * CPT, GCH)], dst_v.at[0])

  def group(g, carry):
    q = lax.rem(g, 2)

    def fire(b, c2):
      pltpu.async_copy(ones_v, cnt_s.at[dst_v.at[q, b]], sem, add=True)
      return c2

    lax.fori_loop(0, GCH, fire, 0)

    @pl.when(g + 1 < ngrp)
    def _():
      base = w * CPT + (g + 1) * GCH
      pltpu.sync_copy(dsti.at[pl.ds(base, GCH)], dst_v.at[1 - q])

    def drain(b, c2):
      pltpu.make_async_copy(ones_v, cnt_s.at[dst_v.at[q, 0]], sem).wait()
      return c2

    lax.fori_loop(0, GCH, drain, 0)
    return carry

  lax.fori_loop(0, ngrp, group, 0)

  plsc.subcore_barrier()
  ob = c * N_PAD + r0
  pltpu.sync_copy(cnt_s.at[pl.ds(r0, RPT)], cnt_out.at[pl.ds(ob, RPT)])


_count = pl.kernel(
    _count_body, mesh=_MESH,
    out_type=[jax.ShapeDtypeStruct((NC * N_PAD, D), jnp.float32)],
    scratch_types=[
        pltpu.VMEM((2, GCH, CHUNK), jnp.int32),
        pltpu.VMEM((CHUNK, D), jnp.float32),
        pltpu.VMEM_SHARED((N_PAD, D), jnp.float32),
        pltpu.SemaphoreType.DMA,
    ])


def _dense1(acc, cnt, x, wl, b, wr):
  def body(a0, a1, c0, c1, x_ref, wl_ref, b_ref, wr_ref, h_ref):
    sums = a0[...] + a1[...]
    counts = jnp.maximum(c0[:, 0:1] + c1[:, 0:1], 1.0)
    agg = sums / counts
    h = (jnp.dot(agg, wl_ref[...], preferred_element_type=jnp.float32)
         + b_ref[...]
         + jnp.dot(x_ref[...], wr_ref[...], preferred_element_type=jnp.float32))
    h_ref[...] = jnp.maximum(h, 0.0)

  return pl.pallas_call(
      body,
      grid=(GRID,),
      in_specs=[
          pl.BlockSpec((ROW_BLK, D), lambda i: (i, 0)),
          pl.BlockSpec((ROW_BLK, D), lambda i: (i + GRID, 0)),
          pl.BlockSpec((ROW_BLK, D), lambda i: (i, 0)),
          pl.BlockSpec((ROW_BLK, D), lambda i: (i + GRID, 0)),
          pl.BlockSpec((ROW_BLK, D), lambda i: (i, 0)),
          pl.BlockSpec((D, D), lambda i: (0, 0)),
          pl.BlockSpec((1, D), lambda i: (0, 0)),
          pl.BlockSpec((D, D), lambda i: (0, 0)),
      ],
      out_specs=pl.BlockSpec((ROW_BLK, D), lambda i: (i, 0)),
      out_shape=jax.ShapeDtypeStruct((N_PAD, D), jnp.float32),
  )(acc, acc, cnt, cnt, x, wl, b, wr)


def _dense2(acc, cnt, h, wl, b, wr):
  def body(a0, a1, c0, c1, h_ref, wl_ref, b_ref, wr_ref, o_ref):
    sums = a0[...] + a1[...]
    counts = jnp.maximum(c0[:, 0:1] + c1[:, 0:1], 1.0)
    agg = sums / counts
    logits = (jnp.dot(agg, wl_ref[...], preferred_element_type=jnp.float32)
              + b_ref[...]
              + jnp.dot(h_ref[...], wr_ref[...], preferred_element_type=jnp.float32))
    m = jnp.max(logits, axis=1, keepdims=True)
    lse = jnp.log(jnp.sum(jnp.exp(logits - m), axis=1, keepdims=True))
    o_ref[...] = logits - m - lse

  return pl.pallas_call(
      body,
      grid=(GRID,),
      in_specs=[
          pl.BlockSpec((ROW_BLK, D), lambda i: (i, 0)),
          pl.BlockSpec((ROW_BLK, D), lambda i: (i + GRID, 0)),
          pl.BlockSpec((ROW_BLK, D), lambda i: (i, 0)),
          pl.BlockSpec((ROW_BLK, D), lambda i: (i + GRID, 0)),
          pl.BlockSpec((ROW_BLK, D), lambda i: (i, 0)),
          pl.BlockSpec((D, N_CLASSES), lambda i: (0, 0)),
          pl.BlockSpec((1, N_CLASSES), lambda i: (0, 0)),
          pl.BlockSpec((D, N_CLASSES), lambda i: (0, 0)),
      ],
      out_specs=pl.BlockSpec((ROW_BLK, N_CLASSES), lambda i: (i, 0)),
      out_shape=jax.ShapeDtypeStruct((N_PAD, N_CLASSES), jnp.float32),
  )(acc, acc, cnt, cnt, h, wl, b, wr)


def kernel(x, edge_index, W1l, b1l, W1r, W2l, b2l, W2r):
  src = edge_index[0].astype(jnp.int32)
  dst = edge_index[1].astype(jnp.int32)
  pad = E_PAD - src.shape[0]
  src2 = jnp.concatenate([src, jnp.zeros((pad,), jnp.int32)]).reshape(-1, CHUNK)
  dst2 = jnp.concatenate([dst, jnp.full((pad,), N_NODES, jnp.int32)]).reshape(-1, CHUNK)
  x_pad = jnp.pad(x, ((0, N_PAD - N_NODES), (0, 0)))
  zer = jnp.zeros((RPT, D), jnp.float32)
  ones = jnp.ones((CHUNK, D), jnp.float32)

  (cnt,) = _count(dst2, zer, ones)
  (acc1,) = _agg(x_pad, src2, dst2, zer)
  h = _dense1(acc1, cnt, x_pad, W1l.T, b1l.reshape(1, D), W1r.T)
  (acc2,) = _agg(h, src2, dst2, zer)
  out = _dense2(acc2, cnt, h, W2l.T, b2l.reshape(1, N_CLASSES), W2r.T)
  return out[:N_NODES]


# speedup vs baseline: 1.0523x; 1.0059x over previous
import jax
import jax.numpy as jnp
from jax import lax
from jax.experimental import pallas as pl
from jax.experimental.pallas import tpu as pltpu
from jax.experimental.pallas import tpu_sc as plsc

N_NODES = 10000
D = 128
N_CLASSES = 40

NC = 2
NS = 16
CHUNK = 128
CPT = 80
GCH = 40
E_PAD = NC * NS * CPT * CHUNK
RPT = 632
N_PAD = NS * RPT
ROW_BLK = RPT
GRID = N_PAD // ROW_BLK

_MESH = plsc.VectorSubcoreMesh(core_axis_name="c", subcore_axis_name="s")


def _agg_body(feat, srci, dsti, zer, acc_out, src_v, dst_v, rows_v, acc_s, sem):
  c = lax.axis_index("c")
  s = lax.axis_index("s")
  w = c * NS + s
  r0 = s * RPT
  pltpu.sync_copy(zer, acc_s.at[pl.ds(r0, RPT)])
  plsc.subcore_barrier()

  def group(g, carry):
    base = w * CPT + g * GCH
    pltpu.sync_copy(srci.at[pl.ds(base, GCH)], src_v)
    pltpu.sync_copy(dsti.at[pl.ds(base, GCH)], dst_v)
    pltpu.async_copy(feat.at[src_v.at[0]], rows_v.at[0], sem)

    def step(b, c2):
      p = lax.rem(b, 2)
      pltpu.make_async_copy(feat.at[src_v.at[b]], rows_v.at[p], sem).wait()

      @pl.when(b + 1 < GCH)
      def _():
        pltpu.async_copy(feat.at[src_v.at[b + 1]], rows_v.at[1 - p], sem)

      pltpu.sync_copy(rows_v.at[p], acc_s.at[dst_v.at[b]], add=True)
      return c2

    lax.fori_loop(0, GCH, step, 0)
    return carry

  lax.fori_loop(0, CPT // GCH, group, 0)

  plsc.subcore_barrier()
  ob = c * N_PAD + r0
  pltpu.sync_copy(acc_s.at[pl.ds(r0, RPT)], acc_out.at[pl.ds(ob, RPT)])


_agg = pl.kernel(
    _agg_body, mesh=_MESH,
    out_type=[jax.ShapeDtypeStruct((NC * N_PAD, D), jnp.float32)],
    scratch_types=[
        pltpu.VMEM((GCH, CHUNK), jnp.int32),
        pltpu.VMEM((GCH, CHUNK), jnp.int32),
        pltpu.VMEM((2, CHUNK, D), jnp.float32),
        pltpu.VMEM_SHARED((N_PAD, D), jnp.float32),
        pltpu.SemaphoreType.DMA,
    ])


def _count_body(dsti, zer, ones, cnt_out, dst_v, ones_v, cnt_s, sem):
  c = lax.axis_index("c")
  s = lax.axis_index("s")
  w = c * NS + s
  r0 = s * RPT
  pltpu.sync_copy(zer, cnt_s.at[pl.ds(r0, RPT)])
  pltpu.sync_copy(ones, ones_v)
  plsc.subcore_barrier()
  ngrp = CPT // GCH
  pltpu.sync_copy(dsti.at[pl.ds(w * CPT, GCH)], dst_v.at[0])

  def group(g, carry):
    q = lax.rem(g, 2)

    def fire(b, c2):
      pltpu.async_copy(ones_v, cnt_s.at[dst_v.at[q, b]], sem, add=True)
      return c2

    lax.fori_loop(0, GCH, fire, 0)

    @pl.when(g + 1 < ngrp)
    def _():
      base = w * CPT + (g + 1) * GCH
      pltpu.sync_copy(dsti.at[pl.ds(base, GCH)], dst_v.at[1 - q])

    def drain(b, c2):
      pltpu.make_async_copy(ones_v, cnt_s.at[dst_v.at[q, 0]], sem).wait()
      return c2

    lax.fori_loop(0, GCH, drain, 0)
    return carry

  lax.fori_loop(0, ngrp, group, 0)

  plsc.subcore_barrier()
  ob = c * N_PAD + r0
  pltpu.sync_copy(cnt_s.at[pl.ds(r0, RPT)], cnt_out.at[pl.ds(ob, RPT)])


_count = pl.kernel(
    _count_body, mesh=_MESH,
    out_type=[jax.ShapeDtypeStruct((NC * N_PAD, D), jnp.float32)],
    scratch_types=[
        pltpu.VMEM((2, GCH, CHUNK), jnp.int32),
        pltpu.VMEM((CHUNK, D), jnp.float32),
        pltpu.VMEM_SHARED((N_PAD, D), jnp.float32),
        pltpu.SemaphoreType.DMA,
    ])


def _dense1(acc, cnt, x, wl, b, wr):
  def body(a0, a1, c0, c1, x_ref, wl_ref, b_ref, wr_ref, h_ref):
    sums = a0[...] + a1[...]
    counts = jnp.maximum(c0[:, 0:1] + c1[:, 0:1], 1.0)
    agg = sums / counts
    h = (jnp.dot(agg, wl_ref[...], preferred_element_type=jnp.float32)
         + b_ref[...]
         + jnp.dot(x_ref[...], wr_ref[...], preferred_element_type=jnp.float32))
    h_ref[...] = jnp.maximum(h, 0.0)

  return pl.pallas_call(
      body,
      grid=(GRID,),
      in_specs=[
          pl.BlockSpec((ROW_BLK, D), lambda i: (i, 0)),
          pl.BlockSpec((ROW_BLK, D), lambda i: (i + GRID, 0)),
          pl.BlockSpec((ROW_BLK, D), lambda i: (i, 0)),
          pl.BlockSpec((ROW_BLK, D), lambda i: (i + GRID, 0)),
          pl.BlockSpec((ROW_BLK, D), lambda i: (i, 0)),
          pl.BlockSpec((D, D), lambda i: (0, 0)),
          pl.BlockSpec((1, D), lambda i: (0, 0)),
          pl.BlockSpec((D, D), lambda i: (0, 0)),
      ],
      out_specs=pl.BlockSpec((ROW_BLK, D), lambda i: (i, 0)),
      out_shape=jax.ShapeDtypeStruct((N_PAD, D), jnp.float32),
  )(acc, acc, cnt, cnt, x, wl, b, wr)


def _dense2(acc, cnt, h, wl, b, wr):
  def body(a0, a1, c0, c1, h_ref, wl_ref, b_ref, wr_ref, o_ref):
    sums = a0[...] + a1[...]
    counts = jnp.maximum(c0[:, 0:1] + c1[:, 0:1], 1.0)
    agg = sums / counts
    logits = (jnp.dot(agg, wl_ref[...], preferred_element_type=jnp.float32)
              + b_ref[...]
              + jnp.dot(h_ref[...], wr_ref[...], preferred_element_type=jnp.float32))
    m = jnp.max(logits, axis=1, keepdims=True)
    lse = jnp.log(jnp.sum(jnp.exp(logits - m), axis=1, keepdims=True))
    o_ref[...] = logits - m - lse

  return pl.pallas_call(
      body,
      grid=(GRID,),
      in_specs=[
          pl.BlockSpec((ROW_BLK, D), lambda i: (i, 0)),
          pl.BlockSpec((ROW_BLK, D), lambda i: (i + GRID, 0)),
          pl.BlockSpec((ROW_BLK, D), lambda i: (i, 0)),
          pl.BlockSpec((ROW_BLK, D), lambda i: (i + GRID, 0)),
          pl.BlockSpec((ROW_BLK, D), lambda i: (i, 0)),
          pl.BlockSpec((D, N_CLASSES), lambda i: (0, 0)),
          pl.BlockSpec((1, N_CLASSES), lambda i: (0, 0)),
          pl.BlockSpec((D, N_CLASSES), lambda i: (0, 0)),
      ],
      out_specs=pl.BlockSpec((ROW_BLK, N_CLASSES), lambda i: (i, 0)),
      out_shape=jax.ShapeDtypeStruct((N_PAD, N_CLASSES), jnp.float32),
  )(acc, acc, cnt, cnt, h, wl, b, wr)


def kernel(x, edge_index, W1l, b1l, W1r, W2l, b2l, W2r):
  src = edge_index[0].astype(jnp.int32)
  dst = edge_index[1].astype(jnp.int32)
  pad = E_PAD - src.shape[0]
  src2 = jnp.concatenate([src, jnp.zeros((pad,), jnp.int32)]).reshape(-1, CHUNK)
  dst2 = jnp.concatenate([dst, jnp.full((pad,), N_NODES, jnp.int32)]).reshape(-1, CHUNK)
  x_pad = jnp.pad(x, ((0, N_PAD - N_NODES), (0, 0)))
  zer = jnp.zeros((RPT, D), jnp.float32)
  ones = jnp.ones((CHUNK, D), jnp.float32)

  (cnt,) = _count(dst2, zer, ones)
  (acc1,) = _agg(x_pad, src2, dst2, zer)
  h = _dense1(acc1, cnt, x_pad, W1l.T, b1l.reshape(1, D), W1r.T)
  (acc2,) = _agg(h, src2, dst2, zer)
  out = _dense2(acc2, cnt, h, W2l.T, b2l.reshape(1, N_CLASSES), W2r.T)
  return out[:N_NODES]
